# Initial kernel scaffold; baseline (speedup 1.0000x reference)
import jax
import jax.numpy as jnp
from jax.experimental import pallas as pl


def kernel(x, edge_attr, u, node2graph, edge2graph, We, Wue, be, Wn, Wun, bn, Wgn_m, Wge_m, bg_m, Wgn_s, Wge_s, bg_s):
    raise NotImplementedError("write your pallas kernel here")


# speedup vs baseline: 10.0498x; 10.0498x over previous
import jax
import jax.numpy as jnp
from jax.experimental import pallas as pl
from jax.experimental.pallas import tpu as pltpu

LOG_SIG_MAX = 2.0
LOG_SIG_MIN = -20.0


def _pick_block(total, target):
    b = min(target, total)
    while total % b:
        b -= 1
    return b


def _edge_body(seg_ref, eblk_ref, u_ref, Wue_ref, be_ref, We_ref,
               acc_ref, cnt_ref, ue_ref):
    i = pl.program_id(0)
    nb = pl.num_programs(0)
    Bg = acc_ref.shape[0]

    @pl.when(i == 0)
    def _init():
        ue_ref[...] = (jnp.dot(u_ref[...], Wue_ref[...],
                               preferred_element_type=jnp.float32)
                       + be_ref[...])
        acc_ref[...] = jnp.zeros_like(acc_ref)
        cnt_ref[...] = jnp.zeros_like(cnt_ref)

    ids = seg_ref[0]
    iota = jax.lax.broadcasted_iota(jnp.int32, (Bg, 1), 0)
    oh = (ids == iota).astype(jnp.float32)
    mm = jnp.dot(eblk_ref[...], We_ref[...],
                 preferred_element_type=jnp.float32)
    ueg = jax.lax.dot_general(oh, ue_ref[...], (((0,), (0,)), ((), ())),
                              preferred_element_type=jnp.float32)
    act = jnp.maximum(mm + ueg, 0.0)
    acc_ref[...] += jnp.dot(oh, act, preferred_element_type=jnp.float32)
    cnt_ref[...] += jnp.sum(oh, axis=1, keepdims=True)


def _node_body(seg_ref, xblk_ref, u_ref, Wun_ref, bn_ref, Wn_ref,
               acc_e_ref, cnt_e_ref,
               Wgn_m_ref, Wge_m_ref, bg_m_ref,
               Wgn_s_ref, Wge_s_ref, bg_s_ref,
               mean_ref, logstd_ref,
               un_ref, acc_ref, cnt_ref):
    i = pl.program_id(0)
    nb = pl.num_programs(0)
    Bg = acc_ref.shape[0]

    @pl.when(i == 0)
    def _init():
        un_ref[...] = (jnp.dot(u_ref[...], Wun_ref[...],
                               preferred_element_type=jnp.float32)
                       + bn_ref[...])
        acc_ref[...] = jnp.zeros_like(acc_ref)
        cnt_ref[...] = jnp.zeros_like(cnt_ref)

    ids = seg_ref[0]
    iota = jax.lax.broadcasted_iota(jnp.int32, (Bg, 1), 0)
    oh = (ids == iota).astype(jnp.float32)
    mm = jnp.dot(xblk_ref[...], Wn_ref[...],
                 preferred_element_type=jnp.float32)
    ung = jax.lax.dot_general(oh, un_ref[...], (((0,), (0,)), ((), ())),
                              preferred_element_type=jnp.float32)
    act = jnp.maximum(mm + ung, 0.0)
    acc_ref[...] += jnp.dot(oh, act, preferred_element_type=jnp.float32)
    cnt_ref[...] += jnp.sum(oh, axis=1, keepdims=True)

    @pl.when(i == nb - 1)
    def _finish():
        n_agg = acc_ref[...] / jnp.maximum(cnt_ref[...], 1.0)
        e_agg = acc_e_ref[...] / jnp.maximum(cnt_e_ref[...], 1.0)
        mean_ref[...] = (
            jnp.dot(n_agg, Wgn_m_ref[...], preferred_element_type=jnp.float32)
            + jnp.dot(e_agg, Wge_m_ref[...], preferred_element_type=jnp.float32)
            + bg_m_ref[...])
        ls = (jnp.dot(n_agg, Wgn_s_ref[...], preferred_element_type=jnp.float32)
              + jnp.dot(e_agg, Wge_s_ref[...], preferred_element_type=jnp.float32)
              + bg_s_ref[...])
        logstd_ref[...] = jnp.clip(ls, LOG_SIG_MIN, LOG_SIG_MAX)


def kernel(x, edge_attr, u, node2graph, edge2graph, We, Wue, be, Wn, Wun, bn,
           Wgn_m, Wge_m, bg_m, Wgn_s, Wge_s, bg_s):
    N, DN = x.shape
    E, DE = edge_attr.shape
    B, DU = u.shape
    H = We.shape[1]
    A = Wgn_m.shape[1]
    f32 = jnp.float32

    BE = _pick_block(E, 4000)
    KE = E // BE
    BN = _pick_block(N, 2000)
    KN = N // BN

    e2g = edge2graph.reshape(KE, 1, BE)
    n2g = node2graph.reshape(KN, 1, BN)
    be2 = be.reshape(1, H)
    bn2 = bn.reshape(1, H)
    bgm2 = bg_m.reshape(1, A)
    bgs2 = bg_s.reshape(1, A)

    full = lambda shape: pl.BlockSpec(shape, lambda i: (0,) * len(shape))

    acc_e, cnt_e = pl.pallas_call(
        _edge_body,
        grid=(KE,),
        in_specs=[
            pl.BlockSpec((1, 1, BE), lambda i: (i, 0, 0)),
            pl.BlockSpec((BE, DE), lambda i: (i, 0)),
            full((B, DU)),
            full((DU, H)),
            full((1, H)),
            full((DE, H)),
        ],
        out_specs=[full((B, H)), full((B, 1))],
        out_shape=[jax.ShapeDtypeStruct((B, H), f32),
                   jax.ShapeDtypeStruct((B, 1), f32)],
        scratch_shapes=[pltpu.VMEM((B, H), f32)],
    )(e2g, edge_attr, u, Wue, be2, We)

    mean, log_std = pl.pallas_call(
        _node_body,
        grid=(KN,),
        in_specs=[
            pl.BlockSpec((1, 1, BN), lambda i: (i, 0, 0)),
            pl.BlockSpec((BN, DN), lambda i: (i, 0)),
            full((B, DU)),
            full((DU, H)),
            full((1, H)),
            full((DN, H)),
            full((B, H)),
            full((B, 1)),
            full((H, A)),
            full((H, A)),
            full((1, A)),
            full((H, A)),
            full((H, A)),
            full((1, A)),
        ],
        out_specs=[full((B, A)), full((B, A))],
        out_shape=[jax.ShapeDtypeStruct((B, A), f32),
                   jax.ShapeDtypeStruct((B, A), f32)],
        scratch_shapes=[pltpu.VMEM((B, H), f32),
                        pltpu.VMEM((B, H), f32),
                        pltpu.VMEM((B, 1), f32)],
    )(n2g, x, u, Wun, bn2, Wn, acc_e, cnt_e,
      Wgn_m, Wge_m, bgm2, Wgn_s, Wge_s, bgs2)

    return (mean, log_std)
